# baseline (speedup 1.0000x reference)
import functools

import jax
import jax.numpy as jnp
from jax import lax
from jax.experimental import pallas as pl
from jax.experimental.pallas import tpu as pltpu
import jax.experimental.pallas.tpu_sc as plsc

N0 = 25000
N1 = 25000
E = 400000
D_IN = 128
D = 64
GCN_LAYERS = 2

NP0 = 25088
NPF = 50176
BT = 3136

NC = 2
NS = 16
CHUNK = 128
NBUF = 4
SEG = 20


def _leaky(v):
    return jnp.where(v >= 0, v, 0.01 * v)


TOTCH = E // CHUNK
SEGSZ = 49


@functools.lru_cache(maxsize=None)
def _make_seg_sum(x_rows, acc_rows, win_lo, win_rows, src_pl, dst_pl,
                  nplanes, src_off=0):
    rpw = win_rows // NS
    mesh = plsc.VectorSubcoreMesh(core_axis_name="c", subcore_axis_name="s")

    @functools.partial(
        pl.kernel,
        out_type=jax.ShapeDtypeStruct((NC * win_rows, D), jnp.bfloat16),
        mesh=mesh,
        scratch_types=[
            pltpu.VMEM((SEGSZ, CHUNK), jnp.int32),
            pltpu.VMEM((SEGSZ, CHUNK), jnp.int32),
            pltpu.VMEM((NBUF, CHUNK, D), jnp.bfloat16),
            pltpu.VMEM_SHARED((acc_rows, D), jnp.bfloat16),
            pltpu.SemaphoreType.DMA,
            pltpu.SemaphoreType.DMA,
        ],
        compiler_params=pltpu.CompilerParams(use_tc_tiling_on_sc=False),
    )
    def seg_sum(x_hbm, ei_hbm, zeros_hbm, out_hbm, sidx, didx, rows, acc,
                sem, ssem):
        c = lax.axis_index("c")
        s = lax.axis_index("s")
        w = c * NS + s
        lo = w * TOTCH // (NC * NS)
        cnt = (w + 1) * TOTCH // (NC * NS) - lo
        pltpu.sync_copy(zeros_hbm.at[pl.ds(s * rpw, rpw)],
                        acc.at[pl.ds(win_lo + s * rpw, rpw)])
        plsc.subcore_barrier()

        def seg_body(t, carry):
            seg_lo = lo + t * SEGSZ
            seg_cnt = jnp.minimum(cnt - t * SEGSZ, SEGSZ)
            pltpu.sync_copy(ei_hbm.at[src_pl, pl.ds(seg_lo, SEGSZ)], sidx)
            pltpu.sync_copy(ei_hbm.at[dst_pl, pl.ds(seg_lo, SEGSZ)], didx)
            if src_off:
                def sub_body(i, carry3):
                    r = i // (CHUNK // 16)
                    l = (i % (CHUNK // 16)) * 16
                    sidx[r, pl.ds(l, 16)] = sidx[r, pl.ds(l, 16)] - src_off
                    return carry3
                lax.fori_loop(0, SEGSZ * (CHUNK // 16), sub_body, 0)
            for b in range(NBUF):
                pltpu.async_copy(x_hbm.at[sidx.at[b]], rows.at[b], sem)

            def body(q, carry2):
                j0 = q * NBUF
                for b in range(NBUF):
                    j = j0 + b

                    @pl.when(j < seg_cnt)
                    def _():
                        pltpu.make_async_copy(x_hbm.at[sidx.at[j]],
                                              rows.at[b], sem).wait()
                        pltpu.async_copy(rows.at[b], acc.at[didx.at[j]],
                                         ssem, add=True)
                for b in range(NBUF):
                    j = j0 + b

                    @pl.when(j < seg_cnt)
                    def _():
                        pltpu.make_async_copy(rows.at[b], acc.at[didx.at[j]],
                                              ssem).wait()

                        @pl.when(j + NBUF < seg_cnt)
                        def _():
                            pltpu.async_copy(x_hbm.at[sidx.at[j + NBUF]],
                                             rows.at[b], sem)
                return carry2

            lax.fori_loop(0, (SEGSZ + NBUF - 1) // NBUF, body, 0)
            return carry

        lax.fori_loop(0, 2, seg_body, 0)
        plsc.subcore_barrier()
        pltpu.sync_copy(acc.at[pl.ds(win_lo + s * rpw, rpw)],
                        out_hbm.at[pl.ds(c * win_rows + s * rpw, rpw)])

    return seg_sum


def _seg_sum(x_bf, ei, src_pl, dst_pl, acc_rows, win_lo, win_rows,
             src_off=0):
    nplanes = ei.shape[0] * ei.shape[1]
    ei_r = ei.reshape(nplanes, TOTCH, CHUNK)
    zeros = jnp.zeros((win_rows, D), jnp.bfloat16)
    return _make_seg_sum(x_bf.shape[0], acc_rows, win_lo, win_rows,
                         src_pl, dst_pl, nplanes, src_off)(x_bf, ei_r, zeros)


def _proj(x, w, b, n_out):
    n, k = x.shape
    m = w.shape[1]

    def body(x_ref, w_ref, b_ref, o_ref):
        o = jnp.dot(x_ref[...], w_ref[...],
                    preferred_element_type=jnp.float32) + b_ref[...]
        o_ref[...] = _leaky(o).astype(jnp.bfloat16)

    return pl.pallas_call(
        body,
        grid=(n_out // BT,),
        in_specs=[pl.BlockSpec((BT, k), lambda i: (i, 0)),
                  pl.BlockSpec((k, m), lambda i: (0, 0)),
                  pl.BlockSpec((1, m), lambda i: (0, 0))],
        out_specs=pl.BlockSpec((BT, m), lambda i: (i, 0)),
        out_shape=jax.ShapeDtypeStruct((n_out, m), jnp.bfloat16),
    )(x, w, b.reshape(1, -1))


def _conv_combine(x_bf, parts, wr, wn, b, g, beta):
    n = parts.shape[0] // 2
    nblk = n // BT

    def body(x_ref, a0_ref, a1_ref, wr_ref, wn_ref, b_ref, g_ref, be_ref,
             o_ref):
        agg = (a0_ref[...].astype(jnp.float32)
               + a1_ref[...].astype(jnp.float32))
        h = (jnp.dot(x_ref[...], wr_ref[...],
                     preferred_element_type=jnp.float32)
             + jnp.dot(agg, wn_ref[...], preferred_element_type=jnp.float32)
             + b_ref[...])
        mu = jnp.mean(h, axis=-1, keepdims=True)
        var = jnp.mean((h - mu) ** 2, axis=-1, keepdims=True)
        o = (h - mu) / jnp.sqrt(var + 1e-5) * g_ref[...] + be_ref[...]
        o_ref[...] = _leaky(o).astype(jnp.bfloat16)

    mat = pl.BlockSpec((BT, D), lambda i: (i, 0))
    par = pl.BlockSpec((D, D), lambda i: (0, 0))
    vec = pl.BlockSpec((1, D), lambda i: (0, 0))
    return pl.pallas_call(
        body,
        grid=(nblk,),
        in_specs=[mat,
                  pl.BlockSpec((BT, D), lambda i: (i, 0)),
                  pl.BlockSpec((BT, D), lambda i: (i + nblk, 0)),
                  par, par, vec, vec, vec],
        out_specs=mat,
        out_shape=jax.ShapeDtypeStruct((n, D), jnp.bfloat16),
    )(x_bf, parts, parts, wr, wn, b.reshape(1, -1),
      g.reshape(1, -1), beta.reshape(1, -1))


def _conv_full(x_bf, parts, h, wr, wn, b, g, beta):
    nblk = NPF // BT
    half = NP0 // BT

    def body(x_ref, a0_ref, a1_ref, h_ref, wr_ref, wn_ref, b_ref, g_ref,
             be_ref, o_ref):
        i = pl.program_id(0)

        @pl.when(i < half)
        def _():
            agg = (a0_ref[...].astype(jnp.float32)
                   + a1_ref[...].astype(jnp.float32))
            hh = (jnp.dot(x_ref[...], wr_ref[...],
                          preferred_element_type=jnp.float32)
                  + jnp.dot(agg, wn_ref[...],
                            preferred_element_type=jnp.float32)
                  + b_ref[...])
            mu = jnp.mean(hh, axis=-1, keepdims=True)
            var = jnp.mean((hh - mu) ** 2, axis=-1, keepdims=True)
            o = (hh - mu) / jnp.sqrt(var + 1e-5) * g_ref[...] + be_ref[...]
            o_ref[...] = _leaky(o).astype(jnp.bfloat16)

        @pl.when(i >= half)
        def _():
            o_ref[...] = h_ref[...]

    clam = lambda i: (jnp.minimum(i, half - 1), 0)
    return pl.pallas_call(
        body,
        grid=(nblk,),
        in_specs=[pl.BlockSpec((BT, D), clam),
                  pl.BlockSpec((BT, D), clam),
                  pl.BlockSpec((BT, D),
                               lambda i: (jnp.minimum(i, half - 1) + half, 0)),
                  pl.BlockSpec((BT, D), lambda i: (i, 0)),
                  pl.BlockSpec((D, D), lambda i: (0, 0)),
                  pl.BlockSpec((D, D), lambda i: (0, 0)),
                  pl.BlockSpec((1, D), lambda i: (0, 0)),
                  pl.BlockSpec((1, D), lambda i: (0, 0)),
                  pl.BlockSpec((1, D), lambda i: (0, 0))],
        out_specs=pl.BlockSpec((BT, D), lambda i: (i, 0)),
        out_shape=jax.ShapeDtypeStruct((NPF, D), jnp.bfloat16),
    )(x_bf, parts, parts, h, wr, wn, b.reshape(1, -1), g.reshape(1, -1),
      beta.reshape(1, -1))


def _att_xs(f1, f2, f3, w, b):
    n = NP0
    k = 3 * D

    def body(f1_ref, f2_ref, f3_ref, w_ref, b_ref, o_ref):
        cat = jnp.concatenate([f1_ref[...], f2_ref[...], f3_ref[...]],
                              axis=-1)
        o = jnp.dot(cat, w_ref[...],
                    preferred_element_type=jnp.float32) + b_ref[...]
        o_ref[...] = _leaky(o).astype(jnp.bfloat16)

    mat = pl.BlockSpec((BT, D), lambda i: (i, 0))
    return pl.pallas_call(
        body,
        grid=(n // BT,),
        in_specs=[mat, mat, mat,
                  pl.BlockSpec((k, k), lambda i: (0, 0)),
                  pl.BlockSpec((1, k), lambda i: (0, 0))],
        out_specs=pl.BlockSpec((BT, k), lambda i: (i, 0)),
        out_shape=jax.ShapeDtypeStruct((n, k), jnp.bfloat16),
    )(f1, f2, f3, w.astype(jnp.bfloat16), b.reshape(1, -1))


def _att_logits(xs, w1, b1, w2, b2, w3, n_valid):
    n, k = xs.shape

    def body(xs_ref, w1_ref, b1_ref, w2_ref, b2_ref, w3_ref, o_ref):
        xv = xs_ref[...]
        t = jnp.tanh(jnp.dot(xv, w1_ref[...],
                             preferred_element_type=jnp.float32) + b1_ref[...])
        s = jax.nn.sigmoid(jnp.dot(xv, w2_ref[...],
                                   preferred_element_type=jnp.float32)
                           + b2_ref[...])
        lg = jnp.dot(t * s, w3_ref[...], preferred_element_type=jnp.float32)
        row = (pl.program_id(0) * BT
               + jax.lax.broadcasted_iota(jnp.int32, (BT, 1), 0))
        o_ref[...] = jnp.where(row < n_valid, lg, -1e30)

    return pl.pallas_call(
        body,
        grid=(n // BT,),
        in_specs=[pl.BlockSpec((BT, k), lambda i: (i, 0)),
                  pl.BlockSpec((k, k), lambda i: (0, 0)),
                  pl.BlockSpec((1, k), lambda i: (0, 0)),
                  pl.BlockSpec((k, k), lambda i: (0, 0)),
                  pl.BlockSpec((1, k), lambda i: (0, 0)),
                  pl.BlockSpec((k, 1), lambda i: (0, 0))],
        out_specs=pl.BlockSpec((BT, 1), lambda i: (i, 0)),
        out_shape=jax.ShapeDtypeStruct((n, 1), jnp.float32),
    )(xs, w1.astype(jnp.bfloat16), b1.reshape(1, -1),
      w2.astype(jnp.bfloat16), b2.reshape(1, -1), w3)


def _softmax_row(l):
    n = l.shape[1]

    def body(l_ref, o_ref):
        lv = l_ref[...]
        m = jnp.max(lv, axis=-1, keepdims=True)
        e = jnp.exp(lv - m)
        o_ref[...] = e / jnp.sum(e, axis=-1, keepdims=True)

    return pl.pallas_call(
        body,
        out_shape=jax.ShapeDtypeStruct((1, n), jnp.float32),
    )(l)


def _pool(atts, xs):
    n, k = xs.shape
    bt = 3584

    def body(a_ref, xs_ref, o_ref):
        @pl.when(pl.program_id(0) == 0)
        def _():
            o_ref[...] = jnp.zeros_like(o_ref)
        o_ref[...] += jnp.dot(a_ref[...], xs_ref[...].astype(jnp.float32),
                              preferred_element_type=jnp.float32)

    return pl.pallas_call(
        body,
        grid=(n // bt,),
        in_specs=[pl.BlockSpec((1, bt), lambda i: (0, i)),
                  pl.BlockSpec((bt, k), lambda i: (i, 0))],
        out_specs=pl.BlockSpec((1, k), lambda i: (0, 0)),
        out_shape=jax.ShapeDtypeStruct((1, k), jnp.float32),
    )(atts, xs)


def _head(xv, wl, bl, wc, bc):
    def body(v_ref, wl_ref, bl_ref, wc_ref, bc_ref, o_ref):
        h1 = _leaky(jnp.dot(v_ref[...], wl_ref[...],
                            preferred_element_type=jnp.float32) + bl_ref[...])
        o_ref[...] = jax.nn.sigmoid(
            jnp.dot(h1, wc_ref[...], preferred_element_type=jnp.float32)
            + bc_ref[...])

    return pl.pallas_call(
        body,
        out_shape=jax.ShapeDtypeStruct((1, 2), jnp.float32),
    )(xv, wl, bl.reshape(1, -1), wc, bc.reshape(1, -1))


def kernel(x, edge_index, edge_index_diff, feats_size_list, mask_prob, params):
    p = params
    h = _proj(x, p['l0_W'], p['l0_b'], NPF)

    ei = edge_index.astype(jnp.int32)
    eid = edge_index_diff.astype(jnp.int32)

    parts = _seg_sum(h, ei, 0, 1, NP0, 0, NP0)
    xx1 = _conv_combine(h, parts, p['g0_0_Wr'], p['g0_0_Wn'], p['g0_0_b'],
                        p['g0_0_ln_g'], p['g0_0_ln_b'])
    parts = _seg_sum(xx1, ei, 0, 1, NP0, 0, NP0)
    full0 = _conv_full(xx1, parts, h, p['g0_1_Wr'], p['g0_1_Wn'],
                       p['g0_1_b'], p['g0_1_ln_g'], p['g0_1_ln_b'])
    cat0 = [h, xx1, full0]

    parts = _seg_sum(full0, eid, 0, 1, NPF, 0, NPF)
    fulld = _conv_combine(full0, parts, p['diff0_Wr'], p['diff0_Wn'],
                          p['diff0_b'], p['diff0_ln_g'], p['diff0_ln_b'])

    s1 = jnp.concatenate(
        [fulld[N0:N0 + N1], jnp.zeros((NP0 - N1, D), jnp.bfloat16)], axis=0)
    parts = _seg_sum(fulld, ei, 2, 3, NPF, N0, NP0)
    xx2 = _conv_combine(s1, parts, p['g1_0_Wr'], p['g1_0_Wn'], p['g1_0_b'],
                        p['g1_0_ln_g'], p['g1_0_ln_b'])
    parts = _seg_sum(xx2, ei, 2, 3, NPF, N0, NP0, src_off=N0)
    xx3 = _conv_combine(xx2, parts, p['g1_1_Wr'], p['g1_1_Wn'], p['g1_1_b'],
                        p['g1_1_ln_g'], p['g1_1_ln_b'])
    cat1 = [s1, xx2, xx3]

    pooled = []
    atts = []
    for i, cat in enumerate([cat0, cat1]):
        xs = _att_xs(cat[0], cat[1], cat[2],
                     p['attl1_%d_W' % i], p['attl1_%d_b' % i])
        logit = _att_logits(xs, p['att1_%d_W' % i], p['att1_%d_b' % i],
                            p['att2_%d_W' % i], p['att2_%d_b' % i],
                            p['att3_%d_W' % i], N0)
        a = _softmax_row(logit.reshape(1, NP0))
        atts.append(a[:, :N0])
        pooled.append(_pool(a, xs))

    x_v = jnp.concatenate(pooled, axis=1)
    x_v = _head(x_v, p['llast_W'], p['llast_b'], p['lcla_W'], p['lcla_b'])
    return (x_v, atts[0], atts[1])


# speedup vs baseline: 1.0268x; 1.0268x over previous
import functools

import jax
import jax.numpy as jnp
from jax import lax
from jax.experimental import pallas as pl
from jax.experimental.pallas import tpu as pltpu
import jax.experimental.pallas.tpu_sc as plsc

N0 = 25000
N1 = 25000
E = 400000
D_IN = 128
D = 64
GCN_LAYERS = 2

NP0 = 25088
NPF = 50176
BT = 3136

NC = 2
NS = 16
CHUNK = 128
NBUF = 4
SEG = 20


def _leaky(v):
    return jnp.where(v >= 0, v, 0.01 * v)


TOTCH = E // CHUNK
SEGSZ = 49


@functools.lru_cache(maxsize=None)
def _make_seg_sum(x_rows, acc_rows, win_lo, win_rows, src_pl, dst_pl,
                  nplanes, src_off=0):
    rpw = win_rows // NS
    mesh = plsc.VectorSubcoreMesh(core_axis_name="c", subcore_axis_name="s")

    @functools.partial(
        pl.kernel,
        out_type=jax.ShapeDtypeStruct((NC * win_rows, D), jnp.bfloat16),
        mesh=mesh,
        scratch_types=[
            pltpu.VMEM((SEGSZ, CHUNK), jnp.int32),
            pltpu.VMEM((SEGSZ, CHUNK), jnp.int32),
            pltpu.VMEM((NBUF, CHUNK, D), jnp.bfloat16),
            pltpu.VMEM_SHARED((acc_rows, D), jnp.bfloat16),
            pltpu.SemaphoreType.DMA,
        ],
        compiler_params=pltpu.CompilerParams(use_tc_tiling_on_sc=False),
    )
    def seg_sum(x_hbm, ei_hbm, zeros_hbm, out_hbm, sidx, didx, rows, acc,
                sem):
        c = lax.axis_index("c")
        s = lax.axis_index("s")
        w = c * NS + s
        lo = w * TOTCH // (NC * NS)
        cnt = (w + 1) * TOTCH // (NC * NS) - lo
        pltpu.sync_copy(zeros_hbm.at[pl.ds(s * rpw, rpw)],
                        acc.at[pl.ds(win_lo + s * rpw, rpw)])
        plsc.subcore_barrier()

        def seg_body(t, carry):
            seg_lo = lo + t * SEGSZ
            seg_cnt = jnp.minimum(cnt - t * SEGSZ, SEGSZ)
            pltpu.sync_copy(ei_hbm.at[src_pl, pl.ds(seg_lo, SEGSZ)], sidx)
            pltpu.sync_copy(ei_hbm.at[dst_pl, pl.ds(seg_lo, SEGSZ)], didx)
            if src_off:
                def sub_body(i, carry3):
                    r = i // (CHUNK // 16)
                    l = (i % (CHUNK // 16)) * 16
                    sidx[r, pl.ds(l, 16)] = sidx[r, pl.ds(l, 16)] - src_off
                    return carry3
                lax.fori_loop(0, SEGSZ * (CHUNK // 16), sub_body, 0)
            for b in range(NBUF):
                pltpu.async_copy(x_hbm.at[sidx.at[b]], rows.at[b], sem)

            def body(j, carry2):
                for b in range(NBUF):
                    @pl.when((j % NBUF == b) & (j < seg_cnt))
                    def _():
                        pltpu.make_async_copy(x_hbm.at[sidx.at[j]],
                                              rows.at[b], sem).wait()
                        pltpu.sync_copy(rows.at[b], acc.at[didx.at[j]],
                                        add=True)

                        @pl.when(j + NBUF < seg_cnt)
                        def _():
                            pltpu.async_copy(x_hbm.at[sidx.at[j + NBUF]],
                                             rows.at[b], sem)
                return carry2

            lax.fori_loop(0, SEGSZ, body, 0)
            return carry

        lax.fori_loop(0, 2, seg_body, 0)
        plsc.subcore_barrier()
        pltpu.sync_copy(acc.at[pl.ds(win_lo + s * rpw, rpw)],
                        out_hbm.at[pl.ds(c * win_rows + s * rpw, rpw)])

    return seg_sum


def _seg_sum(x_bf, ei, src_pl, dst_pl, acc_rows, win_lo, win_rows,
             src_off=0):
    nplanes = ei.shape[0] * ei.shape[1]
    ei_r = ei.reshape(nplanes, TOTCH, CHUNK)
    zeros = jnp.zeros((win_rows, D), jnp.bfloat16)
    return _make_seg_sum(x_bf.shape[0], acc_rows, win_lo, win_rows,
                         src_pl, dst_pl, nplanes, src_off)(x_bf, ei_r, zeros)


def _proj(x, w, b, n_out):
    n, k = x.shape
    m = w.shape[1]

    def body(x_ref, w_ref, b_ref, o_ref):
        o = jnp.dot(x_ref[...], w_ref[...],
                    preferred_element_type=jnp.float32) + b_ref[...]
        o_ref[...] = _leaky(o).astype(jnp.bfloat16)

    return pl.pallas_call(
        body,
        grid=(n_out // BT,),
        in_specs=[pl.BlockSpec((BT, k), lambda i: (i, 0)),
                  pl.BlockSpec((k, m), lambda i: (0, 0)),
                  pl.BlockSpec((1, m), lambda i: (0, 0))],
        out_specs=pl.BlockSpec((BT, m), lambda i: (i, 0)),
        out_shape=jax.ShapeDtypeStruct((n_out, m), jnp.bfloat16),
    )(x, w, b.reshape(1, -1))


def _conv_combine(x_bf, parts, wr, wn, b, g, beta):
    n = parts.shape[0] // 2
    nblk = n // BT

    def body(x_ref, a0_ref, a1_ref, wr_ref, wn_ref, b_ref, g_ref, be_ref,
             o_ref):
        agg = (a0_ref[...].astype(jnp.float32)
               + a1_ref[...].astype(jnp.float32))
        h = (jnp.dot(x_ref[...], wr_ref[...],
                     preferred_element_type=jnp.float32)
             + jnp.dot(agg, wn_ref[...], preferred_element_type=jnp.float32)
             + b_ref[...])
        mu = jnp.mean(h, axis=-1, keepdims=True)
        var = jnp.mean((h - mu) ** 2, axis=-1, keepdims=True)
        o = (h - mu) / jnp.sqrt(var + 1e-5) * g_ref[...] + be_ref[...]
        o_ref[...] = _leaky(o).astype(jnp.bfloat16)

    mat = pl.BlockSpec((BT, D), lambda i: (i, 0))
    par = pl.BlockSpec((D, D), lambda i: (0, 0))
    vec = pl.BlockSpec((1, D), lambda i: (0, 0))
    return pl.pallas_call(
        body,
        grid=(nblk,),
        in_specs=[mat,
                  pl.BlockSpec((BT, D), lambda i: (i, 0)),
                  pl.BlockSpec((BT, D), lambda i: (i + nblk, 0)),
                  par, par, vec, vec, vec],
        out_specs=mat,
        out_shape=jax.ShapeDtypeStruct((n, D), jnp.bfloat16),
    )(x_bf, parts, parts, wr, wn, b.reshape(1, -1),
      g.reshape(1, -1), beta.reshape(1, -1))


def _conv_full(x_bf, parts, h, wr, wn, b, g, beta):
    nblk = NPF // BT
    half = NP0 // BT

    def body(x_ref, a0_ref, a1_ref, h_ref, wr_ref, wn_ref, b_ref, g_ref,
             be_ref, o_ref):
        i = pl.program_id(0)

        @pl.when(i < half)
        def _():
            agg = (a0_ref[...].astype(jnp.float32)
                   + a1_ref[...].astype(jnp.float32))
            hh = (jnp.dot(x_ref[...], wr_ref[...],
                          preferred_element_type=jnp.float32)
                  + jnp.dot(agg, wn_ref[...],
                            preferred_element_type=jnp.float32)
                  + b_ref[...])
            mu = jnp.mean(hh, axis=-1, keepdims=True)
            var = jnp.mean((hh - mu) ** 2, axis=-1, keepdims=True)
            o = (hh - mu) / jnp.sqrt(var + 1e-5) * g_ref[...] + be_ref[...]
            o_ref[...] = _leaky(o).astype(jnp.bfloat16)

        @pl.when(i >= half)
        def _():
            o_ref[...] = h_ref[...]

    clam = lambda i: (jnp.minimum(i, half - 1), 0)
    return pl.pallas_call(
        body,
        grid=(nblk,),
        in_specs=[pl.BlockSpec((BT, D), clam),
                  pl.BlockSpec((BT, D), clam),
                  pl.BlockSpec((BT, D),
                               lambda i: (jnp.minimum(i, half - 1) + half, 0)),
                  pl.BlockSpec((BT, D), lambda i: (i, 0)),
                  pl.BlockSpec((D, D), lambda i: (0, 0)),
                  pl.BlockSpec((D, D), lambda i: (0, 0)),
                  pl.BlockSpec((1, D), lambda i: (0, 0)),
                  pl.BlockSpec((1, D), lambda i: (0, 0)),
                  pl.BlockSpec((1, D), lambda i: (0, 0))],
        out_specs=pl.BlockSpec((BT, D), lambda i: (i, 0)),
        out_shape=jax.ShapeDtypeStruct((NPF, D), jnp.bfloat16),
    )(x_bf, parts, parts, h, wr, wn, b.reshape(1, -1), g.reshape(1, -1),
      beta.reshape(1, -1))


def _att_xs(f1, f2, f3, w, b):
    n = NP0
    k = 3 * D

    def body(f1_ref, f2_ref, f3_ref, w_ref, b_ref, o_ref):
        cat = jnp.concatenate([f1_ref[...], f2_ref[...], f3_ref[...]],
                              axis=-1)
        o = jnp.dot(cat, w_ref[...],
                    preferred_element_type=jnp.float32) + b_ref[...]
        o_ref[...] = _leaky(o).astype(jnp.bfloat16)

    mat = pl.BlockSpec((BT, D), lambda i: (i, 0))
    return pl.pallas_call(
        body,
        grid=(n // BT,),
        in_specs=[mat, mat, mat,
                  pl.BlockSpec((k, k), lambda i: (0, 0)),
                  pl.BlockSpec((1, k), lambda i: (0, 0))],
        out_specs=pl.BlockSpec((BT, k), lambda i: (i, 0)),
        out_shape=jax.ShapeDtypeStruct((n, k), jnp.bfloat16),
    )(f1, f2, f3, w.astype(jnp.bfloat16), b.reshape(1, -1))


def _att_logits(xs, w1, b1, w2, b2, w3, n_valid):
    n, k = xs.shape

    def body(xs_ref, w1_ref, b1_ref, w2_ref, b2_ref, w3_ref, o_ref):
        xv = xs_ref[...]
        t = jnp.tanh(jnp.dot(xv, w1_ref[...],
                             preferred_element_type=jnp.float32) + b1_ref[...])
        s = jax.nn.sigmoid(jnp.dot(xv, w2_ref[...],
                                   preferred_element_type=jnp.float32)
                           + b2_ref[...])
        lg = jnp.dot(t * s, w3_ref[...], preferred_element_type=jnp.float32)
        row = (pl.program_id(0) * BT
               + jax.lax.broadcasted_iota(jnp.int32, (BT, 1), 0))
        o_ref[...] = jnp.where(row < n_valid, lg, -1e30)

    return pl.pallas_call(
        body,
        grid=(n // BT,),
        in_specs=[pl.BlockSpec((BT, k), lambda i: (i, 0)),
                  pl.BlockSpec((k, k), lambda i: (0, 0)),
                  pl.BlockSpec((1, k), lambda i: (0, 0)),
                  pl.BlockSpec((k, k), lambda i: (0, 0)),
                  pl.BlockSpec((1, k), lambda i: (0, 0)),
                  pl.BlockSpec((k, 1), lambda i: (0, 0))],
        out_specs=pl.BlockSpec((BT, 1), lambda i: (i, 0)),
        out_shape=jax.ShapeDtypeStruct((n, 1), jnp.float32),
    )(xs, w1.astype(jnp.bfloat16), b1.reshape(1, -1),
      w2.astype(jnp.bfloat16), b2.reshape(1, -1), w3)


def _softmax_row(l):
    n = l.shape[1]

    def body(l_ref, o_ref):
        lv = l_ref[...]
        m = jnp.max(lv, axis=-1, keepdims=True)
        e = jnp.exp(lv - m)
        o_ref[...] = e / jnp.sum(e, axis=-1, keepdims=True)

    return pl.pallas_call(
        body,
        out_shape=jax.ShapeDtypeStruct((1, n), jnp.float32),
    )(l)


def _pool(atts, xs):
    n, k = xs.shape
    bt = 3584

    def body(a_ref, xs_ref, o_ref):
        @pl.when(pl.program_id(0) == 0)
        def _():
            o_ref[...] = jnp.zeros_like(o_ref)
        o_ref[...] += jnp.dot(a_ref[...], xs_ref[...].astype(jnp.float32),
                              preferred_element_type=jnp.float32)

    return pl.pallas_call(
        body,
        grid=(n // bt,),
        in_specs=[pl.BlockSpec((1, bt), lambda i: (0, i)),
                  pl.BlockSpec((bt, k), lambda i: (i, 0))],
        out_specs=pl.BlockSpec((1, k), lambda i: (0, 0)),
        out_shape=jax.ShapeDtypeStruct((1, k), jnp.float32),
    )(atts, xs)


def _head(xv, wl, bl, wc, bc):
    def body(v_ref, wl_ref, bl_ref, wc_ref, bc_ref, o_ref):
        h1 = _leaky(jnp.dot(v_ref[...], wl_ref[...],
                            preferred_element_type=jnp.float32) + bl_ref[...])
        o_ref[...] = jax.nn.sigmoid(
            jnp.dot(h1, wc_ref[...], preferred_element_type=jnp.float32)
            + bc_ref[...])

    return pl.pallas_call(
        body,
        out_shape=jax.ShapeDtypeStruct((1, 2), jnp.float32),
    )(xv, wl, bl.reshape(1, -1), wc, bc.reshape(1, -1))


def kernel(x, edge_index, edge_index_diff, feats_size_list, mask_prob, params):
    p = params
    h = _proj(x, p['l0_W'], p['l0_b'], NPF)

    ei = edge_index.astype(jnp.int32)
    eid = edge_index_diff.astype(jnp.int32)

    parts = _seg_sum(h, ei, 0, 1, NP0, 0, NP0)
    xx1 = _conv_combine(h, parts, p['g0_0_Wr'], p['g0_0_Wn'], p['g0_0_b'],
                        p['g0_0_ln_g'], p['g0_0_ln_b'])
    parts = _seg_sum(xx1, ei, 0, 1, NP0, 0, NP0)
    full0 = _conv_full(xx1, parts, h, p['g0_1_Wr'], p['g0_1_Wn'],
                       p['g0_1_b'], p['g0_1_ln_g'], p['g0_1_ln_b'])
    cat0 = [h, xx1, full0]

    parts = _seg_sum(full0, eid, 0, 1, NPF, 0, NPF)
    fulld = _conv_combine(full0, parts, p['diff0_Wr'], p['diff0_Wn'],
                          p['diff0_b'], p['diff0_ln_g'], p['diff0_ln_b'])

    s1 = jnp.concatenate(
        [fulld[N0:N0 + N1], jnp.zeros((NP0 - N1, D), jnp.bfloat16)], axis=0)
    parts = _seg_sum(fulld, ei, 2, 3, NPF, N0, NP0)
    xx2 = _conv_combine(s1, parts, p['g1_0_Wr'], p['g1_0_Wn'], p['g1_0_b'],
                        p['g1_0_ln_g'], p['g1_0_ln_b'])
    parts = _seg_sum(xx2, ei, 2, 3, NPF, N0, NP0, src_off=N0)
    xx3 = _conv_combine(xx2, parts, p['g1_1_Wr'], p['g1_1_Wn'], p['g1_1_b'],
                        p['g1_1_ln_g'], p['g1_1_ln_b'])
    cat1 = [s1, xx2, xx3]

    pooled = []
    atts = []
    for i, cat in enumerate([cat0, cat1]):
        xs = _att_xs(cat[0], cat[1], cat[2],
                     p['attl1_%d_W' % i], p['attl1_%d_b' % i])
        logit = _att_logits(xs, p['att1_%d_W' % i], p['att1_%d_b' % i],
                            p['att2_%d_W' % i], p['att2_%d_b' % i],
                            p['att3_%d_W' % i], N0)
        a = _softmax_row(logit.reshape(1, NP0))
        atts.append(a[:, :N0])
        pooled.append(_pool(a, xs))

    x_v = jnp.concatenate(pooled, axis=1)
    x_v = _head(x_v, p['llast_W'], p['llast_b'], p['lcla_W'], p['lcla_b'])
    return (x_v, atts[0], atts[1])
